# baseline (speedup 1.0000x reference)
import functools

import jax
import jax.numpy as jnp
from jax import lax
from jax.experimental import pallas as pl
from jax.experimental.pallas import tpu as pltpu
from jax.experimental.pallas import tpu_sc as plsc

NC = 2
NS = 16
NW = NC * NS
L = 16
JB = 40
TD = 8
TI = 128


@functools.lru_cache(maxsize=None)
def _make(nj, ni, vocab, dim):
    mesh = plsc.VectorSubcoreMesh(core_axis_name="c", subcore_axis_name="s")
    iw = ni // NW
    ntd = dim // TD
    nti = iw // TI
    gi_n = iw // L
    njb = nj // JB

    @functools.partial(
        pl.kernel,
        mesh=mesh,
        out_type=jax.ShapeDtypeStruct((nj, ntd, ni // TI, TD, TI),
                                      jnp.float32),
        compiler_params=pltpu.CompilerParams(
            needs_layout_passes=False, use_tc_tiling_on_sc=True),
        scratch_types=[
            pltpu.VMEM((2, JB, iw), jnp.int32),
            pltpu.VMEM((2, 2, ntd, nti, TD, TI), jnp.float32),
            pltpu.VMEM((dim, vocab), jnp.float32),
            pltpu.SemaphoreType.DMA,
            pltpu.SemaphoreType.DMA,
            pltpu.SemaphoreType.DMA,
            pltpu.SemaphoreType.DMA,
        ],
    )
    def k(idx_hbm, table_hbm, out_hbm, idx_v, rows_v, table_v,
          isem0, isem1, osem0, osem1):
        wid = lax.axis_index("s") * NC + lax.axis_index("c")
        ibase = wid * iw
        isems = (isem0, isem1)
        osems = (osem0, osem1)

        pltpu.sync_copy(table_hbm, table_v)

        def load_idx(jb, b):
            pltpu.async_copy(
                idx_hbm.at[pl.ds(jb * JB, JB), pl.ds(ibase, iw)],
                idx_v.at[b], isems[b])

        def wait_idx(b):
            pltpu.make_async_copy(
                idx_hbm.at[pl.ds(0, JB), pl.ds(ibase, iw)],
                idx_v.at[b], isems[b]).wait()

        def compute(ib, jj, rb, half):
            rows = rows_v.at[rb, half]

            @plsc.parallel_loop(0, gi_n, unroll=4)
            def gi_body(gi):
                idxs = idx_v[ib, jj, pl.ds(gi * L, L)]
                it = gi // (TI // L)
                ii = (gi % (TI // L)) * L

                @plsc.parallel_loop(0, dim, unroll=8)
                def d_body(d):
                    dfull = jnp.full((L,), d, jnp.int32)
                    val = plsc.load_gather(table_v, [dfull, idxs])
                    rows[d // TD, it, d % TD, pl.ds(ii, L)] = val

        def start_out(j, rb):
            pltpu.async_copy(
                rows_v.at[rb],
                out_hbm.at[pl.ds(j, 2), pl.ds(0, ntd), pl.ds(wid * nti, nti)],
                osems[rb])

        def wait_out(rb):
            pltpu.make_async_copy(
                rows_v.at[rb],
                out_hbm.at[pl.ds(0, 2), pl.ds(0, ntd), pl.ds(wid * nti, nti)],
                osems[rb]).wait()

        load_idx(0, 0)
        wait_idx(0)
        for jb in range(njb):
            ib = jb % 2
            if jb + 1 < njb:
                load_idx(jb + 1, 1 - ib)

            def body(t, carry, jb=jb, ib=ib):
                j = jb * JB + 4 * t
                for rb in range(2):
                    if jb == 0:
                        @pl.when(t > 0)
                        def _(rb=rb):
                            wait_out(rb)
                    else:
                        wait_out(rb)
                    compute(ib, 4 * t + 2 * rb, rb, 0)
                    compute(ib, 4 * t + 2 * rb + 1, rb, 1)
                    start_out(j + 2 * rb, rb)
                return carry

            lax.fori_loop(0, JB // 4, body, 0)
            if jb + 1 < njb:
                wait_idx(1 - ib)
        for rb in range(2):
            wait_out(rb)

    return k


def kernel(indices, table):
    n, m = indices.shape
    vocab, dim = table.shape
    idx_t = indices.astype(jnp.int32).T
    table_t = table.T
    out5 = _make(m, n, vocab, dim)(idx_t, table_t)
    out = out5.transpose(2, 4, 0, 1, 3).reshape(n, m, dim)
    return out


# speedup vs baseline: 1.0073x; 1.0073x over previous
import functools

import jax
import jax.numpy as jnp
from jax import lax
from jax.experimental import pallas as pl
from jax.experimental.pallas import tpu as pltpu
from jax.experimental.pallas import tpu_sc as plsc

NC = 2
NS = 16
NW = NC * NS
L = 16
JB = 40
TD = 8
TI = 128


@functools.lru_cache(maxsize=None)
def _make(nj, ni, vocab, dim):
    mesh = plsc.VectorSubcoreMesh(core_axis_name="c", subcore_axis_name="s")
    iw = ni // NW
    ntd = dim // TD
    nti = iw // TI
    gi_n = iw // L
    njb = nj // JB

    @functools.partial(
        pl.kernel,
        mesh=mesh,
        out_type=jax.ShapeDtypeStruct((nj, ntd, ni // TI, TD, TI),
                                      jnp.float32),
        compiler_params=pltpu.CompilerParams(
            needs_layout_passes=False, use_tc_tiling_on_sc=True),
        scratch_types=[
            pltpu.VMEM((2, JB, iw), jnp.int32),
            pltpu.VMEM((2, ntd, nti, TD, TI), jnp.float32),
            pltpu.VMEM((dim, vocab), jnp.float32),
            pltpu.SemaphoreType.DMA,
            pltpu.SemaphoreType.DMA,
            pltpu.SemaphoreType.DMA,
            pltpu.SemaphoreType.DMA,
        ],
    )
    def k(idx_hbm, table_hbm, out_hbm, idx_v, rows_v, table_v,
          isem0, isem1, osem0, osem1):
        wid = lax.axis_index("s") * NC + lax.axis_index("c")
        ibase = wid * iw
        isems = (isem0, isem1)
        osems = (osem0, osem1)

        pltpu.sync_copy(table_hbm, table_v)

        def load_idx(jb, b):
            pltpu.async_copy(
                idx_hbm.at[pl.ds(jb * JB, JB), pl.ds(ibase, iw)],
                idx_v.at[b], isems[b])

        def wait_idx(b):
            pltpu.make_async_copy(
                idx_hbm.at[pl.ds(0, JB), pl.ds(ibase, iw)],
                idx_v.at[b], isems[b]).wait()

        def compute(ib, jj, rb):
            rows = rows_v.at[rb]

            @plsc.parallel_loop(0, gi_n, unroll=4)
            def gi_body(gi):
                idxs = idx_v[ib, jj, pl.ds(gi * L, L)]
                it = gi // (TI // L)
                ii = (gi % (TI // L)) * L

                @plsc.parallel_loop(0, dim, unroll=8)
                def d_body(d):
                    dfull = jnp.full((L,), d, jnp.int32)
                    val = plsc.load_gather(table_v, [dfull, idxs])
                    rows[d // TD, it, d % TD, pl.ds(ii, L)] = val

        def start_out(j, rb):
            pltpu.async_copy(
                rows_v.at[rb],
                out_hbm.at[j, pl.ds(0, ntd), pl.ds(wid * nti, nti)],
                osems[rb])

        def wait_out(rb):
            pltpu.make_async_copy(
                rows_v.at[rb],
                out_hbm.at[0, pl.ds(0, ntd), pl.ds(wid * nti, nti)],
                osems[rb]).wait()

        load_idx(0, 0)
        wait_idx(0)
        for jb in range(njb):
            ib = jb % 2
            if jb + 1 < njb:
                load_idx(jb + 1, 1 - ib)

            def body(t, carry, jb=jb, ib=ib):
                j = jb * JB + 2 * t
                for rb in range(2):
                    if jb == 0:
                        @pl.when(t > 0)
                        def _(rb=rb):
                            wait_out(rb)
                    else:
                        wait_out(rb)
                    compute(ib, 2 * t + rb, rb)
                    start_out(j + rb, rb)
                return carry

            lax.fori_loop(0, JB // 2, body, 0)
            if jb + 1 < njb:
                wait_idx(1 - ib)
        for rb in range(2):
            wait_out(rb)

    return k


def kernel(indices, table):
    n, m = indices.shape
    vocab, dim = table.shape
    idx_t = indices.astype(jnp.int32).T
    table_t = table.T
    out5 = _make(m, n, vocab, dim)(idx_t, table_t)
    out = out5.transpose(2, 4, 0, 1, 3).reshape(n, m, dim)
    return out
